# Initial kernel scaffold; baseline (speedup 1.0000x reference)
import jax
import jax.numpy as jnp
from jax.experimental import pallas as pl


def kernel(shape_ids, color_ids, sym_feats, shape_table, color_table, ln_w, ln_b, fc_w, fc_b):
    raise NotImplementedError("write your pallas kernel here")


# speedup vs baseline: 9.0149x; 9.0149x over previous
import functools

import jax
import jax.numpy as jnp
from jax import lax
from jax.experimental import pallas as pl
from jax.experimental.pallas import tpu as pltpu
from jax.experimental.pallas import tpu_sc as plsc

B, L, D = 4096, 50, 64
NC, NS, LANES = 2, 16, 16
NW = NC * NS
BPT = B // NW
NSL = D // LANES

def _sc_body(sids_hbm, cids_hbm, stab_hbm, ctab_hbm, out_hbm, cnt_hbm,
             sid_v, cid_v, sbufA, cbufA, sbufB, cbufB, acc_v, cnt_v,
             semA, semB):
    wid = lax.axis_index("c") * NS + lax.axis_index("s")

    pltpu.sync_copy(sids_hbm.at[wid], sid_v)
    pltpu.sync_copy(cids_hbm.at[wid], cid_v)

    zf = jnp.zeros((LANES,), jnp.float32)
    of = jnp.ones((LANES,), jnp.float32)
    zi = jnp.zeros((LANES,), jnp.int32)

    for k in range(BPT // LANES):
        cnt_v[pl.ds(LANES * k, LANES)] = zf

    def _zero(r, carry):
        for c in range(NSL):
            acc_v[r, pl.ds(LANES * c, LANES)] = zf
        return carry
    lax.fori_loop(0, BPT, _zero, 0)

    def _premask(g, carry):
        for k in range(BPT // LANES):
            sl = pl.ds(LANES * k, LANES)
            nz = sid_v[g, sl] != 0
            cid_v[g, sl] = jnp.where(nz, cid_v[g, sl], zi)
            plsc.addupdate(cnt_v.at[sl], jnp.where(nz, of, zf))
        return carry
    lax.fori_loop(0, L, _premask, 0)

    def _fire(g, sbuf, cbuf, sem):
        pltpu.async_copy(stab_hbm.at[sid_v.at[g]], sbuf, sem)
        pltpu.async_copy(ctab_hbm.at[cid_v.at[g]], cbuf, sem)

    def _drain(g, sbuf, cbuf, sem):
        pltpu.make_async_copy(stab_hbm.at[sid_v.at[g]], sbuf, sem).wait()
        pltpu.make_async_copy(ctab_hbm.at[cid_v.at[g]], cbuf, sem).wait()

    def _accum(sbuf, cbuf):
        def body(r4, carry):
            for dr in range(4):
                r = r4 * 4 + dr
                for c in range(NSL):
                    sl = pl.ds(LANES * c, LANES)
                    plsc.addupdate(acc_v.at[r, sl], sbuf[r, sl] + cbuf[r, sl])
            return carry
        lax.fori_loop(0, BPT // 4, body, 0)

    _fire(0, sbufA, cbufA, semA)

    def _pipe(i, carry):
        g0 = 2 * i
        _fire(g0 + 1, sbufB, cbufB, semB)
        _drain(g0, sbufA, cbufA, semA)
        _accum(sbufA, cbufA)

        @pl.when(g0 + 2 < L)
        def _():
            _fire(g0 + 2, sbufA, cbufA, semA)

        _drain(g0 + 1, sbufB, cbufB, semB)
        _accum(sbufB, cbufB)
        return carry
    lax.fori_loop(0, L // 2, _pipe, 0)

    pltpu.sync_copy(acc_v, out_hbm.at[pl.ds(wid * BPT, BPT), :])
    pltpu.sync_copy(cnt_v, cnt_hbm.at[pl.ds(wid * BPT, BPT)])


@functools.cache
def _sc_pool():
    mesh = plsc.VectorSubcoreMesh(
        core_axis_name="c", subcore_axis_name="s", num_cores=NC, num_subcores=NS
    )
    return pl.kernel(
        _sc_body,
        out_type=[jax.ShapeDtypeStruct((B, D), jnp.float32),
                  jax.ShapeDtypeStruct((B,), jnp.float32)],
        mesh=mesh,
        scratch_types=[
            pltpu.VMEM((L, BPT), jnp.int32),
            pltpu.VMEM((L, BPT), jnp.int32),
            pltpu.VMEM((BPT, D), jnp.float32),
            pltpu.VMEM((BPT, D), jnp.float32),
            pltpu.VMEM((BPT, D), jnp.float32),
            pltpu.VMEM((BPT, D), jnp.float32),
            pltpu.VMEM((BPT, D), jnp.float32),
            pltpu.VMEM((BPT,), jnp.float32),
            pltpu.SemaphoreType.DMA,
            pltpu.SemaphoreType.DMA,
        ],
        compiler_params=pltpu.CompilerParams(use_tc_tiling_on_sc=False),
    )


def _tc_head_body(pooled_ref, cnt_ref, sym_ref, lnw_ref, lnb_ref, w1_ref,
                  w2_ref, fcb_ref, out_ref):
    s = sym_ref[...]
    mu = (s[:, 0:1] + s[:, 1:2] + s[:, 2:3]) * (1.0 / 3.0)
    d = s - mu
    var = (d[:, 0:1] * d[:, 0:1] + d[:, 1:2] * d[:, 1:2]
           + d[:, 2:3] * d[:, 2:3]) * (1.0 / 3.0)
    inv = lax.rsqrt(var + 1e-5)
    sn = d * inv * lnw_ref[...] + lnb_ref[...]
    pooled = pooled_ref[...] / jnp.maximum(cnt_ref[...], 1.0)
    acc = jnp.dot(pooled, w1_ref[...],
                  preferred_element_type=jnp.float32)
    w2 = w2_ref[...]
    acc = (acc + sn[:, 0:1] * w2[0:1, :] + sn[:, 1:2] * w2[1:2, :]
           + sn[:, 2:3] * w2[2:3, :])
    out_ref[...] = acc + fcb_ref[...]


def kernel(shape_ids, color_ids, sym_feats, shape_table, color_table,
           ln_w, ln_b, fc_w, fc_b):
    nl = fc_w.shape[0]
    sids_t = shape_ids.astype(jnp.int32).reshape(NW, BPT, L).transpose(0, 2, 1)
    cids_t = color_ids.astype(jnp.int32).reshape(NW, BPT, L).transpose(0, 2, 1)

    pooled, cnt = _sc_pool()(sids_t, cids_t, shape_table, color_table)

    w1 = fc_w[:, :D].T
    w2 = fc_w[:, D:].T

    bb = 1024
    grid = (B // bb,)
    logits = pl.pallas_call(
        _tc_head_body,
        grid=grid,
        in_specs=[
            pl.BlockSpec((bb, D), lambda i: (i, 0)),
            pl.BlockSpec((bb, 1), lambda i: (i, 0)),
            pl.BlockSpec((bb, 3), lambda i: (i, 0)),
            pl.BlockSpec((1, 3), lambda i: (0, 0)),
            pl.BlockSpec((1, 3), lambda i: (0, 0)),
            pl.BlockSpec((D, nl), lambda i: (0, 0)),
            pl.BlockSpec((3, nl), lambda i: (0, 0)),
            pl.BlockSpec((1, nl), lambda i: (0, 0)),
        ],
        out_specs=pl.BlockSpec((bb, nl), lambda i: (i, 0)),
        out_shape=jax.ShapeDtypeStruct((B, nl), jnp.float32),
    )(pooled, cnt.reshape(B, 1), sym_feats, ln_w.reshape(1, 3),
      ln_b.reshape(1, 3), w1, w2, fc_b.reshape(1, nl))
    return logits
